# Initial kernel scaffold; baseline (speedup 1.0000x reference)
import jax
import jax.numpy as jnp
from jax.experimental import pallas as pl


def kernel(image, dfeat, neighbor, Egroup_weight, divider, W0, b0, W1, b1, W2, b2, W3, b3):
    raise NotImplementedError("write your pallas kernel here")


# speedup vs baseline: 3.8081x; 3.8081x over previous
import functools

import jax
import jax.numpy as jnp
from jax import lax
from jax.experimental import pallas as pl

NFEAT = 128
H0, H1, H2 = 256, 128, 64


def _mlp_body(x_ref, w0t, b0r, w1t, b1r, w2t, b2r, w3t, b3r, w3row, w2f, w1f, w03,
              etot_ref, ei_ref, de3_ref):
    x = x_ref[0]
    f0 = jnp.dot(x, w0t[...], preferred_element_type=jnp.float32) + b0r[...]
    l0 = jax.nn.softplus(f0)
    d0 = jax.nn.sigmoid(f0)
    f1 = jnp.dot(l0, w1t[...], preferred_element_type=jnp.float32) + b1r[...]
    l1 = jax.nn.softplus(f1)
    d1 = jax.nn.sigmoid(f1)
    f2 = jnp.dot(l1, w2t[...], preferred_element_type=jnp.float32) + b2r[...]
    l2 = jax.nn.softplus(f2)
    d2 = jax.nn.sigmoid(f2)
    ei = jnp.dot(l2, w3t[...], preferred_element_type=jnp.float32) + b3r[...]
    etot_ref[...] = jnp.sum(ei).reshape(1, 1, 1)
    ei_ref[0] = ei
    g = d2 * w3row[...]
    g = jnp.dot(g, w2f[...], preferred_element_type=jnp.float32)
    g = d1 * g
    g = jnp.dot(g, w1f[...], preferred_element_type=jnp.float32)
    g = d0 * g
    de3 = jnp.dot(g, w03[...], preferred_element_type=jnp.float32)
    de3_ref[0] = de3.astype(jnp.bfloat16)


def _force_body(nb_ref, de3_ref, dfeat_ref, out_ref, *, nblk, natom, knb):
    rows = nblk * knb
    nb = nb_ref[0] - 1
    iota = lax.broadcasted_iota(jnp.int32, (nblk, knb, natom), 2)
    oh = (nb[:, :, None] == iota).astype(jnp.bfloat16)
    oh2 = oh.reshape(rows, natom)
    de3 = jnp.dot(oh2, de3_ref[0], preferred_element_type=jnp.float32)
    dfb = dfeat_ref[0].reshape(rows, 3 * NFEAT)
    p = de3 * dfb
    s1 = p.reshape(nblk, knb, 3 * NFEAT).sum(axis=1)
    pp = lax.broadcasted_iota(jnp.int32, (3 * NFEAT, 3), 0)
    dd = lax.broadcasted_iota(jnp.int32, (3 * NFEAT, 3), 1)
    fold = (pp % 3 == dd).astype(jnp.float32)
    out_ref[0] = jnp.dot(s1, fold, preferred_element_type=jnp.float32)


def kernel(image, dfeat, neighbor, Egroup_weight, divider,
           W0, b0, W1, b1, W2, b2, W3, b3):
    B, N, F = image.shape
    K = neighbor.shape[2]
    w0t, w1t, w2t, w3t = W0.T, W1.T, W2.T, W3.T
    b0r, b1r, b2r = b0[None, :], b1[None, :], b2[None, :]
    b3r = b3[None, :]
    w03 = jnp.repeat(W0, 3, axis=1)
    nb = neighbor.astype(jnp.int32)
    dfeat_r = dfeat.reshape(B, N, K, F * 3)

    full = lambda s: pl.BlockSpec(s, lambda b: (0,) * len(s))
    etot, ei, de3 = pl.pallas_call(
        _mlp_body,
        grid=(B,),
        in_specs=[
            pl.BlockSpec((1, N, F), lambda b: (b, 0, 0)),
            full((F, H0)), full((1, H0)),
            full((H0, H1)), full((1, H1)),
            full((H1, H2)), full((1, H2)),
            full((H2, 1)), full((1, 1)),
            full((1, H2)), full((H2, H1)), full((H1, H0)), full((H0, 3 * F)),
        ],
        out_specs=[
            pl.BlockSpec((1, 1, 1), lambda b: (b, 0, 0)),
            pl.BlockSpec((1, N, 1), lambda b: (b, 0, 0)),
            pl.BlockSpec((1, N, 3 * F), lambda b: (b, 0, 0)),
        ],
        out_shape=[
            jax.ShapeDtypeStruct((B, 1, 1), jnp.float32),
            jax.ShapeDtypeStruct((B, N, 1), jnp.float32),
            jax.ShapeDtypeStruct((B, N, 3 * F), jnp.bfloat16),
        ],
    )(image, w0t, b0r, w1t, b1r, w2t, b2r, w3t, b3r, W3, W2, W1, w03)

    NBLK = 8
    force = pl.pallas_call(
        functools.partial(_force_body, nblk=NBLK, natom=N, knb=K),
        grid=(B, N // NBLK),
        in_specs=[
            pl.BlockSpec((1, NBLK, K), lambda b, i: (b, i, 0)),
            pl.BlockSpec((1, N, 3 * F), lambda b, i: (b, 0, 0)),
            pl.BlockSpec((1, NBLK, K, 3 * F), lambda b, i: (b, i, 0, 0)),
        ],
        out_specs=pl.BlockSpec((1, NBLK, 3), lambda b, i: (b, i, 0)),
        out_shape=jax.ShapeDtypeStruct((B, N, 3), jnp.float32),
    )(nb, de3, dfeat_r)

    return (etot.reshape(B, 1), ei, force)


# speedup vs baseline: 4.7460x; 1.2463x over previous
import functools

import jax
import jax.numpy as jnp
from jax import lax
from jax.experimental import pallas as pl

NFEAT = 128
H0, H1, H2 = 256, 128, 64


def _mlp_body(x_ref, w0t, b0r, w1t, b1r, w2t, b2r, w3t, b3r, w3row, w2f, w1f, w03,
              etot_ref, ei_ref, de3_ref):
    x = x_ref[0]
    f0 = jnp.dot(x, w0t[...], preferred_element_type=jnp.float32) + b0r[...]
    l0 = jax.nn.softplus(f0)
    d0 = jax.nn.sigmoid(f0)
    f1 = jnp.dot(l0, w1t[...], preferred_element_type=jnp.float32) + b1r[...]
    l1 = jax.nn.softplus(f1)
    d1 = jax.nn.sigmoid(f1)
    f2 = jnp.dot(l1, w2t[...], preferred_element_type=jnp.float32) + b2r[...]
    l2 = jax.nn.softplus(f2)
    d2 = jax.nn.sigmoid(f2)
    ei = jnp.dot(l2, w3t[...], preferred_element_type=jnp.float32) + b3r[...]
    etot_ref[...] = jnp.sum(ei).reshape(1, 1, 1)
    ei_ref[0] = ei
    g = d2 * w3row[...]
    g = jnp.dot(g, w2f[...], preferred_element_type=jnp.float32)
    g = d1 * g
    g = jnp.dot(g, w1f[...], preferred_element_type=jnp.float32)
    g = d0 * g
    de3 = jnp.dot(g, w03[...], preferred_element_type=jnp.float32)
    de3_ref[0] = de3.astype(jnp.bfloat16)


def _force_body(nb_ref, de3_ref, dfeat_ref, out_ref, *, nblk, natom, knb):
    rows = nblk * knb
    nb = nb_ref[0] - 1
    iota = lax.broadcasted_iota(jnp.int32, (nblk, knb, natom), 2)
    oh = (nb[:, :, None] == iota).astype(jnp.bfloat16)
    oh2 = oh.reshape(rows, natom)
    de3 = jnp.dot(oh2, de3_ref[0], preferred_element_type=jnp.float32)
    dfb = dfeat_ref[0].reshape(rows, 3 * NFEAT).astype(jnp.float32)
    p = de3 * dfb
    s1 = p.reshape(nblk, knb, 3 * NFEAT).sum(axis=1)
    pp = lax.broadcasted_iota(jnp.int32, (3 * NFEAT, 3), 0)
    dd = lax.broadcasted_iota(jnp.int32, (3 * NFEAT, 3), 1)
    fold = (pp % 3 == dd).astype(jnp.float32)
    out_ref[0] = jnp.dot(s1, fold, preferred_element_type=jnp.float32)


def kernel(image, dfeat, neighbor, Egroup_weight, divider,
           W0, b0, W1, b1, W2, b2, W3, b3):
    B, N, F = image.shape
    K = neighbor.shape[2]
    w0t, w1t, w2t, w3t = W0.T, W1.T, W2.T, W3.T
    b0r, b1r, b2r = b0[None, :], b1[None, :], b2[None, :]
    b3r = b3[None, :]
    w03 = jnp.repeat(W0, 3, axis=1)
    nb = neighbor.astype(jnp.int32)
    dfeat_r = dfeat.reshape(B, N, K, F * 3).astype(jnp.bfloat16)

    full = lambda s: pl.BlockSpec(s, lambda b: (0,) * len(s))
    etot, ei, de3 = pl.pallas_call(
        _mlp_body,
        grid=(B,),
        in_specs=[
            pl.BlockSpec((1, N, F), lambda b: (b, 0, 0)),
            full((F, H0)), full((1, H0)),
            full((H0, H1)), full((1, H1)),
            full((H1, H2)), full((1, H2)),
            full((H2, 1)), full((1, 1)),
            full((1, H2)), full((H2, H1)), full((H1, H0)), full((H0, 3 * F)),
        ],
        out_specs=[
            pl.BlockSpec((1, 1, 1), lambda b: (b, 0, 0)),
            pl.BlockSpec((1, N, 1), lambda b: (b, 0, 0)),
            pl.BlockSpec((1, N, 3 * F), lambda b: (b, 0, 0)),
        ],
        out_shape=[
            jax.ShapeDtypeStruct((B, 1, 1), jnp.float32),
            jax.ShapeDtypeStruct((B, N, 1), jnp.float32),
            jax.ShapeDtypeStruct((B, N, 3 * F), jnp.bfloat16),
        ],
    )(image, w0t, b0r, w1t, b1r, w2t, b2r, w3t, b3r, W3, W2, W1, w03)

    NBLK = 16
    force = pl.pallas_call(
        functools.partial(_force_body, nblk=NBLK, natom=N, knb=K),
        grid=(B, N // NBLK),
        in_specs=[
            pl.BlockSpec((1, NBLK, K), lambda b, i: (b, i, 0)),
            pl.BlockSpec((1, N, 3 * F), lambda b, i: (b, 0, 0)),
            pl.BlockSpec((1, NBLK, K, 3 * F), lambda b, i: (b, i, 0, 0)),
        ],
        out_specs=pl.BlockSpec((1, NBLK, 3), lambda b, i: (b, i, 0)),
        out_shape=jax.ShapeDtypeStruct((B, N, 3), jnp.float32),
    )(nb, de3, dfeat_r)

    return (etot.reshape(B, 1), ei, force)


# speedup vs baseline: 13.9188x; 2.9327x over previous
import functools

import jax
import jax.numpy as jnp
from jax import lax
from jax.experimental import pallas as pl

NFEAT = 128
H0, H1, H2 = 256, 128, 64


def _mlp_body(x_ref, w0t, b0r, w1t, b1r, w2t, b2r, w3t, b3r, w3row, w2f, w1f, w0f,
              etot_ref, ei_ref, de_ref):
    x = x_ref[0]
    f0 = jnp.dot(x, w0t[...], preferred_element_type=jnp.float32) + b0r[...]
    l0 = jax.nn.softplus(f0)
    d0 = jax.nn.sigmoid(f0)
    f1 = jnp.dot(l0, w1t[...], preferred_element_type=jnp.float32) + b1r[...]
    l1 = jax.nn.softplus(f1)
    d1 = jax.nn.sigmoid(f1)
    f2 = jnp.dot(l1, w2t[...], preferred_element_type=jnp.float32) + b2r[...]
    l2 = jax.nn.softplus(f2)
    d2 = jax.nn.sigmoid(f2)
    ei = jnp.dot(l2, w3t[...], preferred_element_type=jnp.float32) + b3r[...]
    etot_ref[...] = jnp.sum(ei).reshape(1, 1, 1)
    ei_ref[0] = ei
    g = d2 * w3row[...]
    g = jnp.dot(g, w2f[...], preferred_element_type=jnp.float32)
    g = d1 * g
    g = jnp.dot(g, w1f[...], preferred_element_type=jnp.float32)
    g = d0 * g
    de = jnp.dot(g, w0f[...], preferred_element_type=jnp.float32)
    de_ref[0] = de.astype(jnp.bfloat16)


def _force_body(nb_ref, de_ref, dft_ref, out_ref, *, nblk, natom, knb):
    rows = nblk * knb
    nb = nb_ref[0] - 1
    iota = lax.broadcasted_iota(jnp.int32, (nblk, knb, natom), 2)
    oh = (nb[:, :, None] == iota).astype(jnp.bfloat16)
    oh2 = oh.reshape(rows, natom)
    de_nb = jnp.dot(oh2, de_ref[0], preferred_element_type=jnp.float32)
    de3 = de_nb.reshape(nblk, 1, knb, NFEAT)
    dft = dft_ref[0]
    p = de3 * dft
    s = p.sum(axis=2).sum(axis=2)
    out_ref[0] = s


def kernel(image, dfeat, neighbor, Egroup_weight, divider,
           W0, b0, W1, b1, W2, b2, W3, b3):
    B, N, F = image.shape
    K = neighbor.shape[2]
    w0t, w1t, w2t, w3t = W0.T, W1.T, W2.T, W3.T
    b0r, b1r, b2r = b0[None, :], b1[None, :], b2[None, :]
    b3r = b3[None, :]
    nb = neighbor.astype(jnp.int32)
    dfeat_t = dfeat.transpose(0, 1, 4, 2, 3)

    full = lambda s: pl.BlockSpec(s, lambda b: (0,) * len(s))
    etot, ei, de = pl.pallas_call(
        _mlp_body,
        grid=(B,),
        in_specs=[
            pl.BlockSpec((1, N, F), lambda b: (b, 0, 0)),
            full((F, H0)), full((1, H0)),
            full((H0, H1)), full((1, H1)),
            full((H1, H2)), full((1, H2)),
            full((H2, 1)), full((1, 1)),
            full((1, H2)), full((H2, H1)), full((H1, H0)), full((H0, F)),
        ],
        out_specs=[
            pl.BlockSpec((1, 1, 1), lambda b: (b, 0, 0)),
            pl.BlockSpec((1, N, 1), lambda b: (b, 0, 0)),
            pl.BlockSpec((1, N, F), lambda b: (b, 0, 0)),
        ],
        out_shape=[
            jax.ShapeDtypeStruct((B, 1, 1), jnp.float32),
            jax.ShapeDtypeStruct((B, N, 1), jnp.float32),
            jax.ShapeDtypeStruct((B, N, F), jnp.bfloat16),
        ],
    )(image, w0t, b0r, w1t, b1r, w2t, b2r, w3t, b3r, W3, W2, W1, W0)

    NBLK = 16
    force = pl.pallas_call(
        functools.partial(_force_body, nblk=NBLK, natom=N, knb=K),
        grid=(B, N // NBLK),
        in_specs=[
            pl.BlockSpec((1, NBLK, K), lambda b, i: (b, i, 0)),
            pl.BlockSpec((1, N, F), lambda b, i: (b, 0, 0)),
            pl.BlockSpec((1, NBLK, 3, K, F), lambda b, i: (b, i, 0, 0, 0)),
        ],
        out_specs=pl.BlockSpec((1, NBLK, 3), lambda b, i: (b, i, 0)),
        out_shape=jax.ShapeDtypeStruct((B, N, 3), jnp.float32),
    )(nb, de, dfeat_t)

    return (etot.reshape(B, 1), ei, force)


# speedup vs baseline: 18.8488x; 1.3542x over previous
import functools

import jax
import jax.numpy as jnp
from jax import lax
from jax.experimental import pallas as pl

NFEAT = 128
H0, H1, H2 = 256, 128, 64


def _mlp_body(x_ref, w0t, b0r, w1t, b1r, w2t, b2r, w3t, b3r, w3row, w2f, w1f, w0f,
              etot_ref, ei_ref, de_ref):
    x = x_ref[0]
    f0 = jnp.dot(x, w0t[...], preferred_element_type=jnp.float32) + b0r[...]
    l0 = jax.nn.softplus(f0)
    d0 = jax.nn.sigmoid(f0)
    f1 = jnp.dot(l0, w1t[...], preferred_element_type=jnp.float32) + b1r[...]
    l1 = jax.nn.softplus(f1)
    d1 = jax.nn.sigmoid(f1)
    f2 = jnp.dot(l1, w2t[...], preferred_element_type=jnp.float32) + b2r[...]
    l2 = jax.nn.softplus(f2)
    d2 = jax.nn.sigmoid(f2)
    ei = jnp.dot(l2, w3t[...], preferred_element_type=jnp.float32) + b3r[...]
    etot_ref[...] = jnp.sum(ei).reshape(1, 1, 1)
    ei_ref[0] = ei
    g = d2 * w3row[...]
    g = jnp.dot(g, w2f[...], preferred_element_type=jnp.float32)
    g = d1 * g
    g = jnp.dot(g, w1f[...], preferred_element_type=jnp.float32)
    g = d0 * g
    de = jnp.dot(g, w0f[...], preferred_element_type=jnp.float32)
    de_ref[0] = de.astype(jnp.bfloat16)


def _force_body(nb_ref, de_ref, dft_ref, out_ref, *, nblk, natom, knb):
    rows = nblk * knb
    nb = nb_ref[0] - 1
    iota = lax.broadcasted_iota(jnp.int32, (nblk, knb, natom), 2)
    oh = (nb[:, :, None] == iota).astype(jnp.bfloat16)
    oh2 = oh.reshape(rows, natom)
    de_nb = jnp.dot(oh2, de_ref[0], preferred_element_type=jnp.float32)
    de3 = de_nb.reshape(nblk, 1, knb, NFEAT)
    dft = dft_ref[0]
    p = de3 * dft
    s = p.sum(axis=2).sum(axis=2)
    out_ref[0] = s


def kernel(image, dfeat, neighbor, Egroup_weight, divider,
           W0, b0, W1, b1, W2, b2, W3, b3):
    B, N, F = image.shape
    K = neighbor.shape[2]
    w0t, w1t, w2t, w3t = W0.T, W1.T, W2.T, W3.T
    b0r, b1r, b2r = b0[None, :], b1[None, :], b2[None, :]
    b3r = b3[None, :]
    nb = neighbor.astype(jnp.int32)
    dfeat_t = dfeat.transpose(0, 1, 4, 2, 3)

    full = lambda s: pl.BlockSpec(s, lambda b: (0,) * len(s))
    etot, ei, de = pl.pallas_call(
        _mlp_body,
        grid=(B,),
        in_specs=[
            pl.BlockSpec((1, N, F), lambda b: (b, 0, 0)),
            full((F, H0)), full((1, H0)),
            full((H0, H1)), full((1, H1)),
            full((H1, H2)), full((1, H2)),
            full((H2, 1)), full((1, 1)),
            full((1, H2)), full((H2, H1)), full((H1, H0)), full((H0, F)),
        ],
        out_specs=[
            pl.BlockSpec((1, 1, 1), lambda b: (b, 0, 0)),
            pl.BlockSpec((1, N, 1), lambda b: (b, 0, 0)),
            pl.BlockSpec((1, N, F), lambda b: (b, 0, 0)),
        ],
        out_shape=[
            jax.ShapeDtypeStruct((B, 1, 1), jnp.float32),
            jax.ShapeDtypeStruct((B, N, 1), jnp.float32),
            jax.ShapeDtypeStruct((B, N, F), jnp.bfloat16),
        ],
    )(image, w0t, b0r, w1t, b1r, w2t, b2r, w3t, b3r, W3, W2, W1, W0)

    NBLK = 32
    force = pl.pallas_call(
        functools.partial(_force_body, nblk=NBLK, natom=N, knb=K),
        grid=(B, N // NBLK),
        in_specs=[
            pl.BlockSpec((1, NBLK, K), lambda b, i: (b, i, 0)),
            pl.BlockSpec((1, N, F), lambda b, i: (b, 0, 0)),
            pl.BlockSpec((1, NBLK, 3, K, F), lambda b, i: (b, i, 0, 0, 0)),
        ],
        out_specs=pl.BlockSpec((1, NBLK, 3), lambda b, i: (b, i, 0)),
        out_shape=jax.ShapeDtypeStruct((B, N, 3), jnp.float32),
    )(nb, de, dfeat_t)

    return (etot.reshape(B, 1), ei, force)


# speedup vs baseline: 23.0088x; 1.2207x over previous
import functools

import jax
import jax.numpy as jnp
from jax import lax
from jax.experimental import pallas as pl

NFEAT = 128
H0, H1, H2 = 256, 128, 64


def _mlp_body(x_ref, w0t, b0r, w1t, b1r, w2t, b2r, w3t, b3r, w3row, w2f, w1f, w0f,
              etot_ref, ei_ref, de_ref):
    x = x_ref[0]
    f0 = jnp.dot(x, w0t[...], preferred_element_type=jnp.float32) + b0r[...]
    l0 = jax.nn.softplus(f0)
    d0 = jax.nn.sigmoid(f0)
    f1 = jnp.dot(l0, w1t[...], preferred_element_type=jnp.float32) + b1r[...]
    l1 = jax.nn.softplus(f1)
    d1 = jax.nn.sigmoid(f1)
    f2 = jnp.dot(l1, w2t[...], preferred_element_type=jnp.float32) + b2r[...]
    l2 = jax.nn.softplus(f2)
    d2 = jax.nn.sigmoid(f2)
    ei = jnp.dot(l2, w3t[...], preferred_element_type=jnp.float32) + b3r[...]
    etot_ref[...] = jnp.sum(ei).reshape(1, 1, 1)
    ei_ref[0] = ei
    g = d2 * w3row[...]
    g = jnp.dot(g, w2f[...], preferred_element_type=jnp.float32)
    g = d1 * g
    g = jnp.dot(g, w1f[...], preferred_element_type=jnp.float32)
    g = d0 * g
    de = jnp.dot(g, w0f[...], preferred_element_type=jnp.float32)
    de_ref[0] = de.astype(jnp.bfloat16)


def _force_body(nb_ref, de_ref, dft_ref, out_ref, *, nblk, natom, knb):
    rows = nblk * knb
    nb = nb_ref[0] - 1
    iota = lax.broadcasted_iota(jnp.int32, (nblk, knb, natom), 2)
    oh = (nb[:, :, None] == iota).astype(jnp.bfloat16)
    oh2 = oh.reshape(rows, natom)
    de_nb = jnp.dot(oh2, de_ref[0], preferred_element_type=jnp.float32)
    de3 = de_nb.reshape(nblk, 1, knb, NFEAT)
    dft = dft_ref[0]
    p = de3 * dft
    s = p.sum(axis=2).sum(axis=2)
    out_ref[0] = s


def kernel(image, dfeat, neighbor, Egroup_weight, divider,
           W0, b0, W1, b1, W2, b2, W3, b3):
    B, N, F = image.shape
    K = neighbor.shape[2]
    w0t, w1t, w2t, w3t = W0.T, W1.T, W2.T, W3.T
    b0r, b1r, b2r = b0[None, :], b1[None, :], b2[None, :]
    b3r = b3[None, :]
    nb = neighbor.astype(jnp.int32)
    dfeat_t = dfeat.transpose(0, 1, 4, 2, 3)

    full = lambda s: pl.BlockSpec(s, lambda b: (0,) * len(s))
    etot, ei, de = pl.pallas_call(
        _mlp_body,
        grid=(B,),
        in_specs=[
            pl.BlockSpec((1, N, F), lambda b: (b, 0, 0)),
            full((F, H0)), full((1, H0)),
            full((H0, H1)), full((1, H1)),
            full((H1, H2)), full((1, H2)),
            full((H2, 1)), full((1, 1)),
            full((1, H2)), full((H2, H1)), full((H1, H0)), full((H0, F)),
        ],
        out_specs=[
            pl.BlockSpec((1, 1, 1), lambda b: (b, 0, 0)),
            pl.BlockSpec((1, N, 1), lambda b: (b, 0, 0)),
            pl.BlockSpec((1, N, F), lambda b: (b, 0, 0)),
        ],
        out_shape=[
            jax.ShapeDtypeStruct((B, 1, 1), jnp.float32),
            jax.ShapeDtypeStruct((B, N, 1), jnp.float32),
            jax.ShapeDtypeStruct((B, N, F), jnp.bfloat16),
        ],
    )(image, w0t, b0r, w1t, b1r, w2t, b2r, w3t, b3r, W3, W2, W1, W0)

    NBLK = 64
    force = pl.pallas_call(
        functools.partial(_force_body, nblk=NBLK, natom=N, knb=K),
        grid=(B, N // NBLK),
        in_specs=[
            pl.BlockSpec((1, NBLK, K), lambda b, i: (b, i, 0)),
            pl.BlockSpec((1, N, F), lambda b, i: (b, 0, 0)),
            pl.BlockSpec((1, NBLK, 3, K, F), lambda b, i: (b, i, 0, 0, 0)),
        ],
        out_specs=pl.BlockSpec((1, NBLK, 3), lambda b, i: (b, i, 0)),
        out_shape=jax.ShapeDtypeStruct((B, N, 3), jnp.float32),
    )(nb, de, dfeat_t)

    return (etot.reshape(B, 1), ei, force)


# speedup vs baseline: 23.9127x; 1.0393x over previous
import functools

import jax
import jax.numpy as jnp
from jax import lax
from jax.experimental import pallas as pl

NFEAT = 128
H0, H1, H2 = 256, 128, 64


def _mlp_body(x_ref, w0t, b0r, w1t, b1r, w2t, b2r, w3t, b3r, w3row, w2f, w1f, w0f,
              etot_ref, ei_ref, de_ref):
    x = x_ref[0]
    f0 = jnp.dot(x, w0t[...], preferred_element_type=jnp.float32) + b0r[...]
    l0 = jax.nn.softplus(f0)
    d0 = jax.nn.sigmoid(f0)
    f1 = jnp.dot(l0, w1t[...], preferred_element_type=jnp.float32) + b1r[...]
    l1 = jax.nn.softplus(f1)
    d1 = jax.nn.sigmoid(f1)
    f2 = jnp.dot(l1, w2t[...], preferred_element_type=jnp.float32) + b2r[...]
    l2 = jax.nn.softplus(f2)
    d2 = jax.nn.sigmoid(f2)
    ei = jnp.dot(l2, w3t[...], preferred_element_type=jnp.float32) + b3r[...]
    etot_ref[...] = jnp.sum(ei).reshape(1, 1, 1)
    ei_ref[0] = ei
    g = d2 * w3row[...]
    g = jnp.dot(g, w2f[...], preferred_element_type=jnp.float32)
    g = d1 * g
    g = jnp.dot(g, w1f[...], preferred_element_type=jnp.float32)
    g = d0 * g
    de = jnp.dot(g, w0f[...], preferred_element_type=jnp.float32)
    de_ref[0] = de.astype(jnp.bfloat16)


def _force_body(nb_ref, de_ref, dft_ref, out_ref, *, nblk, natom, knb):
    rows = nblk * knb
    nb = nb_ref[0] - 1
    iota = lax.broadcasted_iota(jnp.int32, (nblk, knb, natom), 2)
    oh = (nb[:, :, None] == iota).astype(jnp.bfloat16)
    oh2 = oh.reshape(rows, natom)
    de_nb = jnp.dot(oh2, de_ref[0], preferred_element_type=jnp.float32)
    de3 = de_nb.reshape(nblk, 1, knb, NFEAT)
    dft = dft_ref[0]
    p = de3 * dft
    s = p.sum(axis=2).sum(axis=2)
    out_ref[0] = s


def kernel(image, dfeat, neighbor, Egroup_weight, divider,
           W0, b0, W1, b1, W2, b2, W3, b3):
    B, N, F = image.shape
    K = neighbor.shape[2]
    w0t, w1t, w2t, w3t = W0.T, W1.T, W2.T, W3.T
    b0r, b1r, b2r = b0[None, :], b1[None, :], b2[None, :]
    b3r = b3[None, :]
    nb = neighbor.astype(jnp.int32)
    dfeat_t = dfeat.transpose(0, 1, 4, 2, 3)

    full = lambda s: pl.BlockSpec(s, lambda b: (0,) * len(s))
    etot, ei, de = pl.pallas_call(
        _mlp_body,
        grid=(B,),
        in_specs=[
            pl.BlockSpec((1, N, F), lambda b: (b, 0, 0)),
            full((F, H0)), full((1, H0)),
            full((H0, H1)), full((1, H1)),
            full((H1, H2)), full((1, H2)),
            full((H2, 1)), full((1, 1)),
            full((1, H2)), full((H2, H1)), full((H1, H0)), full((H0, F)),
        ],
        out_specs=[
            pl.BlockSpec((1, 1, 1), lambda b: (b, 0, 0)),
            pl.BlockSpec((1, N, 1), lambda b: (b, 0, 0)),
            pl.BlockSpec((1, N, F), lambda b: (b, 0, 0)),
        ],
        out_shape=[
            jax.ShapeDtypeStruct((B, 1, 1), jnp.float32),
            jax.ShapeDtypeStruct((B, N, 1), jnp.float32),
            jax.ShapeDtypeStruct((B, N, F), jnp.bfloat16),
        ],
    )(image, w0t, b0r, w1t, b1r, w2t, b2r, w3t, b3r, W3, W2, W1, W0)

    NBLK = 128
    force = pl.pallas_call(
        functools.partial(_force_body, nblk=NBLK, natom=N, knb=K),
        grid=(B, N // NBLK),
        in_specs=[
            pl.BlockSpec((1, NBLK, K), lambda b, i: (b, i, 0)),
            pl.BlockSpec((1, N, F), lambda b, i: (b, 0, 0)),
            pl.BlockSpec((1, NBLK, 3, K, F), lambda b, i: (b, i, 0, 0, 0)),
        ],
        out_specs=pl.BlockSpec((1, NBLK, 3), lambda b, i: (b, i, 0)),
        out_shape=jax.ShapeDtypeStruct((B, N, 3), jnp.float32),
    )(nb, de, dfeat_t)

    return (etot.reshape(B, 1), ei, force)
